# baseline (speedup 1.0000x reference)
import functools

import jax
import jax.numpy as jnp
from jax import lax
from jax.experimental import pallas as pl
from jax.experimental.pallas import tpu as pltpu
from jax.experimental.pallas import tpu_sc as plsc

N = 10000
E = 320000
D = 128
H = 128
S = 2000
G = 64
C = 10
EPS = 1e-5

NC = 2
NS = 16
NW = NC * NS

NPAD = 10240
DEGW = 128
EW = E // NW
ECH = 80
ENCH = 127
NU = N + 16
PROWS = NPAD // NW
PCH = 64
PNCH = PROWS // PCH
SPAD = 2048

_mesh = plsc.VectorSubcoreMesh(core_axis_name="c", subcore_axis_name="s")


def _wid():
    return lax.axis_index("c") * NS + lax.axis_index("s")


@functools.partial(
    pl.kernel,
    out_type=jax.ShapeDtypeStruct((NC, NPAD, DEGW), jnp.float32),
    mesh=_mesh,
    scratch_types=[
        pltpu.VMEM((ENCH, ECH), jnp.int32),
        pltpu.VMEM((ECH, DEGW), jnp.float32),
        pltpu.VMEM((ECH, DEGW), jnp.float32),
        pltpu.VMEM_SHARED((NPAD, DEGW), jnp.float32),
        pltpu.SemaphoreType.DMA,
    ],
)
def _deg_sc(dst_hbm, ones_hbm, zeros_hbm, out_hbm, dst_v, ones_v, zz_v, acc,
            sem):
    cid = lax.axis_index("c")
    sid = lax.axis_index("s")
    pltpu.sync_copy(dst_hbm.at[_wid()], dst_v)
    pltpu.sync_copy(ones_hbm, ones_v)
    pltpu.sync_copy(zeros_hbm, zz_v)
    for k in range(NPAD // NS // ECH):
        pltpu.sync_copy(zz_v, acc.at[pl.ds(sid * (NPAD // NS) + k * ECH, ECH)])
    plsc.subcore_barrier()

    def fire(j, carry):
        pltpu.async_copy(ones_v, acc.at[dst_v.at[j]], sem, add=True)
        return carry

    lax.fori_loop(0, ENCH, fire, 0)

    def drain(j, carry):
        pltpu.make_async_copy(ones_v, acc.at[dst_v.at[0]], sem).wait()
        return carry

    lax.fori_loop(0, ENCH, drain, 0)
    plsc.subcore_barrier()
    for k in range(NPAD // NS // ECH):
        r0 = sid * (NPAD // NS) + k * ECH
        pltpu.sync_copy(acc.at[pl.ds(r0, ECH)], zz_v)
        pltpu.sync_copy(zz_v, out_hbm.at[cid, pl.ds(r0, ECH)])


@functools.partial(
    pl.kernel,
    out_type=jax.ShapeDtypeStruct((NC, NPAD, H), jnp.float32),
    mesh=_mesh,
    scratch_types=[
        [pltpu.VMEM((2, ECH), jnp.int32)] * 4,
        [pltpu.VMEM((ECH,), jnp.int32)] * 4,
        [pltpu.VMEM((ECH, H), jnp.float32)] * 4,
        pltpu.VMEM_SHARED((NPAD, H), jnp.float32),
        [pltpu.SemaphoreType.DMA] * 4,
        [pltpu.SemaphoreType.DMA] * 4,
        [pltpu.SemaphoreType.DMA] * 4,
    ],
)
def _prop_sc(u_hbm, ed_hbm, zeros_hbm, out_hbm, ch, dsc, rows, acc,
             semg, sems, semi):
    cid = lax.axis_index("c")
    sid = lax.axis_index("s")
    wid = _wid()

    pltpu.sync_copy(zeros_hbm, rows[0])
    for k in range(NPAD // NS // ECH):
        pltpu.sync_copy(rows[0], acc.at[pl.ds(sid * (NPAD // NS) + k * ECH, ECH)])
    plsc.subcore_barrier()

    def cp_dst(k):
        for v in range(ECH // 16):
            dsc[k][pl.ds(16 * v, 16)] = ch[k][1, pl.ds(16 * v, 16)]

    def finish(k):
        pltpu.make_async_copy(u_hbm.at[ch[k].at[0]], rows[k], semg[k]).wait()
        cp_dst(k)
        pltpu.async_copy(rows[k], acc.at[dsc[k]], sems[k], add=True)

    pltpu.sync_copy(ed_hbm.at[wid, 0], ch[0])
    pltpu.async_copy(u_hbm.at[ch[0].at[0]], rows[0], semg[0])
    for k in (1, 2, 3):
        pltpu.async_copy(ed_hbm.at[wid, k], ch[k], semi[k])
    for j in (0, 1, 2):
        kn = j + 1
        pltpu.make_async_copy(ed_hbm.at[wid, 0], ch[kn], semi[kn]).wait()
        pltpu.async_copy(u_hbm.at[ch[kn].at[0]], rows[kn], semg[kn])
        finish(j)
        pltpu.async_copy(ed_hbm.at[wid, j + 4], ch[j], semi[j])

    def body(oj, carry):
        for b in range(4):
            j = 4 * oj + 3 + b
            k = (3 + b) % 4
            kn = (k + 1) % 4
            pltpu.make_async_copy(ed_hbm.at[wid, 0], ch[kn], semi[kn]).wait()
            pltpu.make_async_copy(rows[kn], acc.at[dsc[kn]], sems[kn]).wait()
            pltpu.async_copy(u_hbm.at[ch[kn].at[0]], rows[kn], semg[kn])
            finish(k)
            jn = jnp.minimum(j + 4, ENCH - 1)
            pltpu.async_copy(ed_hbm.at[wid, jn], ch[k], semi[k])
        return carry

    lax.fori_loop(0, (ENCH - 3) // 4, body, 0)
    pltpu.make_async_copy(u_hbm.at[ch[3].at[0]], rows[3], semg[3]).wait()
    for k in (0, 1, 2):
        pltpu.make_async_copy(ed_hbm.at[wid, 0], ch[k], semi[k]).wait()
        pltpu.make_async_copy(rows[k], acc.at[dsc[k]], sems[k]).wait()

    plsc.subcore_barrier()
    for k in range(NPAD // NS // ECH):
        r0 = sid * (NPAD // NS) + k * ECH
        pltpu.sync_copy(acc.at[pl.ds(r0, ECH)], rows[0])
        pltpu.sync_copy(rows[0], out_hbm.at[cid, pl.ds(r0, ECH)])


@functools.partial(
    pl.kernel,
    out_type=[jax.ShapeDtypeStruct((NC, SPAD, H), jnp.float32)] * 3,
    mesh=_mesh,
    scratch_types=[
        [pltpu.VMEM((PCH,), jnp.int32)] * 2,
        [[pltpu.VMEM((PCH, H), jnp.float32)] * 3] * 2,
        pltpu.VMEM((SPAD // NS, H), jnp.float32),
        pltpu.VMEM_SHARED((SPAD, H), jnp.float32),
        pltpu.VMEM_SHARED((SPAD, H), jnp.float32),
        pltpu.VMEM_SHARED((SPAD, H), jnp.float32),
        [pltpu.SemaphoreType.DMA] * 2,
        [pltpu.SemaphoreType.DMA] * 2,
    ],
)
def _pool_sc(x1_hbm, x2_hbm, x3_hbm, n2s_hbm, zeros_hbm,
             o1_hbm, o2_hbm, o3_hbm,
             idx_v, bufs, zb_v, a1, a2, a3, seml, sems):
    cid = lax.axis_index("c")
    sid = lax.axis_index("s")
    accs = (a1, a2, a3)
    xs = (x1_hbm, x2_hbm, x3_hbm)
    zrows = SPAD // NS
    pltpu.sync_copy(zeros_hbm, zb_v)
    for acc in accs:
        pltpu.sync_copy(zb_v, acc.at[pl.ds(sid * zrows, zrows)])
    plsc.subcore_barrier()

    rb = _wid() * PROWS

    def fire_loads(j, p):
        pltpu.async_copy(n2s_hbm.at[pl.ds(rb + j * PCH, PCH)], idx_v[p], seml[p])
        for i in range(3):
            pltpu.async_copy(xs[i].at[pl.ds(rb + j * PCH, PCH)], bufs[p][i],
                             seml[p])

    def wait_loads(p):
        pltpu.make_async_copy(n2s_hbm.at[pl.ds(rb, PCH)], idx_v[p],
                              seml[p]).wait()
        for i in range(3):
            pltpu.make_async_copy(xs[i].at[pl.ds(rb, PCH)], bufs[p][i],
                                  seml[p]).wait()

    def wait_scats(p):
        for i in range(3):
            pltpu.make_async_copy(bufs[p][i], accs[i].at[idx_v[p]],
                                  sems[p]).wait()

    fire_loads(0, 0)
    fire_loads(1, 1)
    for j in range(PNCH):
        p = j % 2
        wait_loads(p)
        for i in range(3):
            pltpu.async_copy(bufs[p][i], accs[i].at[idx_v[p]], sems[p], add=True)
        if j + 2 < PNCH:
            wait_scats(p)
            fire_loads(j + 2, p)
    wait_scats((PNCH - 2) % 2)
    wait_scats((PNCH - 1) % 2)

    plsc.subcore_barrier()
    for acc, out in ((a1, o1_hbm), (a2, o2_hbm), (a3, o3_hbm)):
        pltpu.sync_copy(acc.at[pl.ds(sid * zrows, zrows)], zb_v)
        pltpu.sync_copy(zb_v, out.at[cid, pl.ds(sid * zrows, zrows)])


def _bn_relu(y, g, b):
    mu = jnp.mean(y, axis=0, keepdims=True)
    yc = y - mu
    var = jnp.mean(yc * yc, axis=0, keepdims=True)
    return jax.nn.relu(yc * lax.rsqrt(var + EPS) * g + b)


def _tc_pre_body(x_ref, w_ref, degp_ref, u_ref, dinv_ref):
    deg = (degp_ref[0, :, 0].astype(jnp.float32)
           + degp_ref[1, :, 0].astype(jnp.float32) + 1.0)
    dinv = lax.rsqrt(deg[:N])[:, None]
    dinv_ref[...] = dinv
    u = jnp.dot(x_ref[...], w_ref[...],
                preferred_element_type=jnp.float32) * dinv
    u_ref[...] = jnp.concatenate(
        [u, jnp.zeros((NU - N, H), jnp.float32)], axis=0)


def _tc_pre(x, w, degp):
    return pl.pallas_call(
        _tc_pre_body,
        out_shape=[
            jax.ShapeDtypeStruct((NU, H), jnp.float32),
            jax.ShapeDtypeStruct((N, 1), jnp.float32),
        ],
    )(x, w, degp)


def _tc_mid_body(tp_ref, u_ref, dinv_ref, b_ref, g_ref, be_ref, wn_ref,
                 x_ref, un_ref):
    dinv = dinv_ref[...]
    y = dinv * (tp_ref[0, :N] + tp_ref[1, :N] + u_ref[:N]) + b_ref[...]
    xl = _bn_relu(y, g_ref[...], be_ref[...])
    x_ref[...] = xl
    un = jnp.dot(xl, wn_ref[...], preferred_element_type=jnp.float32) * dinv
    un_ref[...] = jnp.concatenate(
        [un, jnp.zeros((NU - N, H), jnp.float32)], axis=0)


def _tc_mid(tp, u, dinv, b, g, be, wn):
    return pl.pallas_call(
        _tc_mid_body,
        out_shape=[
            jax.ShapeDtypeStruct((N, H), jnp.float32),
            jax.ShapeDtypeStruct((NU, H), jnp.float32),
        ],
    )(tp, u, dinv, b[None, :], g[None, :], be[None, :], wn)


def _tc_last_body(tp_ref, u_ref, dinv_ref, b_ref, g_ref, be_ref, x_ref):
    y = dinv_ref[...] * (tp_ref[0, :N] + tp_ref[1, :N] + u_ref[:N]) + b_ref[...]
    x_ref[...] = _bn_relu(y, g_ref[...], be_ref[...])


def _tc_last(tp, u, dinv, b, g, be):
    return pl.pallas_call(
        _tc_last_body,
        out_shape=jax.ShapeDtypeStruct((N, H), jnp.float32),
    )(tp, u, dinv, b[None, :], g[None, :], be[None, :])


def _tc_final_body(p1_ref, p2_ref, p3_ref, s2g_ref, gl_ref, bel_ref, w1_ref,
                   b1_ref, w2_ref, b2_ref, out_ref):
    xp = jnp.concatenate(
        [p1_ref[0, :S] + p1_ref[1, :S],
         p2_ref[0, :S] + p2_ref[1, :S],
         p3_ref[0, :S] + p3_ref[1, :S]], axis=1)
    gids = lax.broadcasted_iota(jnp.int32, (S, G), 1)
    onehot = (s2g_ref[...][:, None] == gids).astype(jnp.float32)
    sg = lax.dot_general(onehot, xp, (((0,), (0,)), ((), ())),
                         preferred_element_type=jnp.float32)
    cnt = jnp.sum(onehot, axis=0)
    xg = sg / jnp.maximum(cnt, 1.0)[:, None]
    h = jnp.dot(xg, w1_ref[...], preferred_element_type=jnp.float32) + b1_ref[...]
    h = _bn_relu(h, gl_ref[...], bel_ref[...])
    logits = jnp.dot(h, w2_ref[...], preferred_element_type=jnp.float32) + b2_ref[...]
    m = jnp.max(logits, axis=-1, keepdims=True)
    lse = m + jnp.log(jnp.sum(jnp.exp(logits - m), axis=-1, keepdims=True))
    out_ref[...] = logits - lse


def _tc_final(pp, s2g, gl, bel, w1, b1, w2, b2):
    return pl.pallas_call(
        _tc_final_body,
        out_shape=jax.ShapeDtypeStruct((G, C), jnp.float32),
    )(pp[0], pp[1], pp[2], s2g, gl[None, :], bel[None, :], w1, b1[None, :],
      w2, b2[None, :])


def kernel(x, edge_index, node_to_subgraph, subgraph_to_graph, batch,
           W1, b1, W2, b2, W3, b3, g1, g2, g3, gl, be1, be2, be3, bel,
           lin1_W, lin1_b, lin2_W, lin2_b):
    npad_e = ENCH * ECH - EW
    pad_src = jnp.broadcast_to(
        N + (jnp.arange(npad_e, dtype=jnp.int32) % 16), (NW, npad_e))
    pad_dst = pad_src + 16
    ei = edge_index.astype(jnp.int32).reshape(2, NW, EW)
    srcp = jnp.concatenate([ei[0], pad_src], axis=1).reshape(NW, ENCH, ECH)
    dstp = jnp.concatenate([ei[1], pad_dst], axis=1).reshape(NW, ENCH, ECH)
    ed = jnp.stack([srcp, dstp], axis=2)
    dst = dstp
    n2s = node_to_subgraph.astype(jnp.int32)
    s2g = subgraph_to_graph.astype(jnp.int32)

    ones_deg = jnp.ones((ECH, DEGW), jnp.float32)
    z_deg = jnp.zeros((ECH, DEGW), jnp.float32)
    z_prop = jnp.zeros((ECH, H), jnp.float32)
    z_pool = jnp.zeros((SPAD // NS, H), jnp.float32)

    degp = _deg_sc(dst, ones_deg, z_deg)
    u1, dinv = _tc_pre(x, W1, degp)
    t1 = _prop_sc(u1, ed, z_prop)
    x1, u2 = _tc_mid(t1, u1, dinv, b1, g1, be1, W2)
    t2 = _prop_sc(u2, ed, z_prop)
    x2, u3 = _tc_mid(t2, u2, dinv, b2, g2, be2, W3)
    t3 = _prop_sc(u3, ed, z_prop)
    x3 = _tc_last(t3, u3, dinv, b3, g3, be3)

    zrow = jnp.zeros((NPAD - N, H), jnp.float32)
    n2sp = jnp.concatenate([n2s, jnp.zeros((NPAD - N,), jnp.int32)])
    pp = _pool_sc(jnp.concatenate([x1, zrow], axis=0),
                  jnp.concatenate([x2, zrow], axis=0),
                  jnp.concatenate([x3, zrow], axis=0),
                  n2sp, z_pool)
    return _tc_final(pp, s2g, gl, bel, lin1_W, lin1_b, lin2_W, lin2_b)


# speedup vs baseline: 1.1787x; 1.1787x over previous
import functools

import jax
import jax.numpy as jnp
from jax import lax
from jax.experimental import pallas as pl
from jax.experimental.pallas import tpu as pltpu
from jax.experimental.pallas import tpu_sc as plsc

N = 10000
E = 320000
D = 128
H = 128
S = 2000
G = 64
C = 10
EPS = 1e-5

NC = 2
NS = 16
NW = NC * NS

NPAD = 10240
DEGW = 128
EW = E // NW
ECH = 80
ENCH = EW // ECH
NU = N
PROWS = NPAD // NW
PCH = 64
PNCH = PROWS // PCH
SPAD = 2048

_mesh = plsc.VectorSubcoreMesh(core_axis_name="c", subcore_axis_name="s")


def _wid():
    return lax.axis_index("c") * NS + lax.axis_index("s")


@functools.partial(
    pl.kernel,
    out_type=jax.ShapeDtypeStruct((NC, NPAD, DEGW), jnp.float32),
    mesh=_mesh,
    scratch_types=[
        pltpu.VMEM((ENCH, ECH), jnp.int32),
        pltpu.VMEM((ECH, DEGW), jnp.float32),
        pltpu.VMEM((ECH, DEGW), jnp.float32),
        pltpu.VMEM_SHARED((NPAD, DEGW), jnp.float32),
        pltpu.SemaphoreType.DMA,
    ],
)
def _deg_sc(dst_hbm, ones_hbm, zeros_hbm, out_hbm, dst_v, ones_v, zz_v, acc,
            sem):
    cid = lax.axis_index("c")
    sid = lax.axis_index("s")
    pltpu.sync_copy(dst_hbm.at[_wid()], dst_v)
    pltpu.sync_copy(ones_hbm, ones_v)
    pltpu.sync_copy(zeros_hbm, zz_v)
    for k in range(NPAD // NS // ECH):
        pltpu.sync_copy(zz_v, acc.at[pl.ds(sid * (NPAD // NS) + k * ECH, ECH)])
    plsc.subcore_barrier()

    def fire(j, carry):
        pltpu.async_copy(ones_v, acc.at[dst_v.at[j]], sem, add=True)
        return carry

    lax.fori_loop(0, ENCH, fire, 0)

    def drain(j, carry):
        pltpu.make_async_copy(ones_v, acc.at[dst_v.at[0]], sem).wait()
        return carry

    lax.fori_loop(0, ENCH, drain, 0)
    plsc.subcore_barrier()
    for k in range(NPAD // NS // ECH):
        r0 = sid * (NPAD // NS) + k * ECH
        pltpu.sync_copy(acc.at[pl.ds(r0, ECH)], zz_v)
        pltpu.sync_copy(zz_v, out_hbm.at[cid, pl.ds(r0, ECH)])


@functools.partial(
    pl.kernel,
    out_type=jax.ShapeDtypeStruct((NC, NPAD, H), jnp.float32),
    mesh=_mesh,
    scratch_types=[
        [pltpu.VMEM((2, ECH), jnp.int32)] * 3,
        [pltpu.VMEM((ECH,), jnp.int32)] * 3,
        [pltpu.VMEM((ECH, H), jnp.float32)] * 3,
        pltpu.VMEM_SHARED((NPAD, H), jnp.float32),
        [pltpu.SemaphoreType.DMA] * 3,
        [pltpu.SemaphoreType.DMA] * 3,
        [pltpu.SemaphoreType.DMA] * 3,
    ],
)
def _prop_sc(u_hbm, ed_hbm, zeros_hbm, out_hbm, ch, dsc, rows, acc,
             semg, sems, semi):
    cid = lax.axis_index("c")
    sid = lax.axis_index("s")
    wid = _wid()

    pltpu.sync_copy(zeros_hbm, rows[0])
    for k in range(NPAD // NS // ECH):
        pltpu.sync_copy(rows[0], acc.at[pl.ds(sid * (NPAD // NS) + k * ECH, ECH)])
    plsc.subcore_barrier()

    def cp_dst(k):
        for v in range(ECH // 16):
            dsc[k][pl.ds(16 * v, 16)] = ch[k][1, pl.ds(16 * v, 16)]

    def finish(k):
        pltpu.make_async_copy(u_hbm.at[ch[k].at[0]], rows[k], semg[k]).wait()
        cp_dst(k)
        pltpu.async_copy(rows[k], acc.at[dsc[k]], sems[k], add=True)

    pltpu.sync_copy(ed_hbm.at[wid, 0], ch[0])
    pltpu.async_copy(u_hbm.at[ch[0].at[0]], rows[0], semg[0])
    for k in (1, 2):
        pltpu.async_copy(ed_hbm.at[wid, k], ch[k], semi[k])
    for j in (0, 1):
        kn = j + 1
        pltpu.make_async_copy(ed_hbm.at[wid, 0], ch[kn], semi[kn]).wait()
        pltpu.async_copy(u_hbm.at[ch[kn].at[0]], rows[kn], semg[kn])
        finish(j)
        pltpu.async_copy(ed_hbm.at[wid, j + 3], ch[j], semi[j])

    def body(oj, carry):
        for b in range(3):
            j = 3 * oj + 2 + b
            k = (2 + b) % 3
            kn = (k + 1) % 3
            pltpu.make_async_copy(ed_hbm.at[wid, 0], ch[kn], semi[kn]).wait()
            pltpu.make_async_copy(rows[kn], acc.at[dsc[kn]], sems[kn]).wait()
            pltpu.async_copy(u_hbm.at[ch[kn].at[0]], rows[kn], semg[kn])
            finish(k)
            jn = jnp.minimum(j + 3, ENCH - 1)
            pltpu.async_copy(ed_hbm.at[wid, jn], ch[k], semi[k])
        return carry

    lax.fori_loop(0, (ENCH - 2) // 3, body, 0)
    pltpu.make_async_copy(u_hbm.at[ch[2].at[0]], rows[2], semg[2]).wait()
    for k in (0, 1):
        pltpu.make_async_copy(ed_hbm.at[wid, 0], ch[k], semi[k]).wait()
        pltpu.make_async_copy(rows[k], acc.at[dsc[k]], sems[k]).wait()

    plsc.subcore_barrier()
    for k in range(NPAD // NS // ECH):
        r0 = sid * (NPAD // NS) + k * ECH
        pltpu.sync_copy(acc.at[pl.ds(r0, ECH)], rows[0])
        pltpu.sync_copy(rows[0], out_hbm.at[cid, pl.ds(r0, ECH)])


@functools.partial(
    pl.kernel,
    out_type=[jax.ShapeDtypeStruct((NC, SPAD, H), jnp.float32)] * 3,
    mesh=_mesh,
    scratch_types=[
        [pltpu.VMEM((PCH,), jnp.int32)] * 2,
        [[pltpu.VMEM((PCH, H), jnp.float32)] * 3] * 2,
        pltpu.VMEM((SPAD // NS, H), jnp.float32),
        pltpu.VMEM_SHARED((SPAD, H), jnp.float32),
        pltpu.VMEM_SHARED((SPAD, H), jnp.float32),
        pltpu.VMEM_SHARED((SPAD, H), jnp.float32),
        [pltpu.SemaphoreType.DMA] * 2,
        [pltpu.SemaphoreType.DMA] * 2,
    ],
)
def _pool_sc(x1_hbm, x2_hbm, x3_hbm, n2s_hbm, zeros_hbm,
             o1_hbm, o2_hbm, o3_hbm,
             idx_v, bufs, zb_v, a1, a2, a3, seml, sems):
    cid = lax.axis_index("c")
    sid = lax.axis_index("s")
    accs = (a1, a2, a3)
    xs = (x1_hbm, x2_hbm, x3_hbm)
    zrows = SPAD // NS
    pltpu.sync_copy(zeros_hbm, zb_v)
    for acc in accs:
        pltpu.sync_copy(zb_v, acc.at[pl.ds(sid * zrows, zrows)])
    plsc.subcore_barrier()

    rb = _wid() * PROWS

    def fire_loads(j, p):
        pltpu.async_copy(n2s_hbm.at[pl.ds(rb + j * PCH, PCH)], idx_v[p], seml[p])
        for i in range(3):
            pltpu.async_copy(xs[i].at[pl.ds(rb + j * PCH, PCH)], bufs[p][i],
                             seml[p])

    def wait_loads(p):
        pltpu.make_async_copy(n2s_hbm.at[pl.ds(rb, PCH)], idx_v[p],
                              seml[p]).wait()
        for i in range(3):
            pltpu.make_async_copy(xs[i].at[pl.ds(rb, PCH)], bufs[p][i],
                                  seml[p]).wait()

    def wait_scats(p):
        for i in range(3):
            pltpu.make_async_copy(bufs[p][i], accs[i].at[idx_v[p]],
                                  sems[p]).wait()

    fire_loads(0, 0)
    fire_loads(1, 1)
    for j in range(PNCH):
        p = j % 2
        wait_loads(p)
        for i in range(3):
            pltpu.async_copy(bufs[p][i], accs[i].at[idx_v[p]], sems[p], add=True)
        if j + 2 < PNCH:
            wait_scats(p)
            fire_loads(j + 2, p)
    wait_scats((PNCH - 2) % 2)
    wait_scats((PNCH - 1) % 2)

    plsc.subcore_barrier()
    for acc, out in ((a1, o1_hbm), (a2, o2_hbm), (a3, o3_hbm)):
        pltpu.sync_copy(acc.at[pl.ds(sid * zrows, zrows)], zb_v)
        pltpu.sync_copy(zb_v, out.at[cid, pl.ds(sid * zrows, zrows)])


def _pad_u(u):
    if NU == N:
        return u
    return jnp.concatenate([u, jnp.zeros((NU - N, H), jnp.float32)], axis=0)


def _bn_relu(y, g, b):
    mu = jnp.mean(y, axis=0, keepdims=True)
    yc = y - mu
    var = jnp.mean(yc * yc, axis=0, keepdims=True)
    return jax.nn.relu(yc * lax.rsqrt(var + EPS) * g + b)


def _tc_pre_body(x_ref, w_ref, degp_ref, u_ref, dinv_ref):
    deg = (degp_ref[0, :, 0].astype(jnp.float32)
           + degp_ref[1, :, 0].astype(jnp.float32) + 1.0)
    dinv = lax.rsqrt(deg[:N])[:, None]
    dinv_ref[...] = dinv
    u = jnp.dot(x_ref[...], w_ref[...],
                preferred_element_type=jnp.float32) * dinv
    u_ref[...] = _pad_u(u)


def _tc_pre(x, w, degp):
    return pl.pallas_call(
        _tc_pre_body,
        out_shape=[
            jax.ShapeDtypeStruct((NU, H), jnp.float32),
            jax.ShapeDtypeStruct((N, 1), jnp.float32),
        ],
    )(x, w, degp)


def _tc_mid_body(tp_ref, u_ref, dinv_ref, b_ref, g_ref, be_ref, wn_ref,
                 x_ref, un_ref):
    dinv = dinv_ref[...]
    y = dinv * (tp_ref[0, :N] + tp_ref[1, :N] + u_ref[:N]) + b_ref[...]
    xl = _bn_relu(y, g_ref[...], be_ref[...])
    x_ref[...] = xl
    un = jnp.dot(xl, wn_ref[...], preferred_element_type=jnp.float32) * dinv
    un_ref[...] = _pad_u(un)


def _tc_mid(tp, u, dinv, b, g, be, wn):
    return pl.pallas_call(
        _tc_mid_body,
        out_shape=[
            jax.ShapeDtypeStruct((N, H), jnp.float32),
            jax.ShapeDtypeStruct((NU, H), jnp.float32),
        ],
    )(tp, u, dinv, b[None, :], g[None, :], be[None, :], wn)


def _tc_last_body(tp_ref, u_ref, dinv_ref, b_ref, g_ref, be_ref, x_ref):
    y = dinv_ref[...] * (tp_ref[0, :N] + tp_ref[1, :N] + u_ref[:N]) + b_ref[...]
    x_ref[...] = _bn_relu(y, g_ref[...], be_ref[...])


def _tc_last(tp, u, dinv, b, g, be):
    return pl.pallas_call(
        _tc_last_body,
        out_shape=jax.ShapeDtypeStruct((N, H), jnp.float32),
    )(tp, u, dinv, b[None, :], g[None, :], be[None, :])


def _tc_final_body(p1_ref, p2_ref, p3_ref, s2g_ref, gl_ref, bel_ref, w1_ref,
                   b1_ref, w2_ref, b2_ref, out_ref):
    xp = jnp.concatenate(
        [p1_ref[0, :S] + p1_ref[1, :S],
         p2_ref[0, :S] + p2_ref[1, :S],
         p3_ref[0, :S] + p3_ref[1, :S]], axis=1)
    gids = lax.broadcasted_iota(jnp.int32, (S, G), 1)
    onehot = (s2g_ref[...][:, None] == gids).astype(jnp.float32)
    sg = lax.dot_general(onehot, xp, (((0,), (0,)), ((), ())),
                         preferred_element_type=jnp.float32)
    cnt = jnp.sum(onehot, axis=0)
    xg = sg / jnp.maximum(cnt, 1.0)[:, None]
    h = jnp.dot(xg, w1_ref[...], preferred_element_type=jnp.float32) + b1_ref[...]
    h = _bn_relu(h, gl_ref[...], bel_ref[...])
    logits = jnp.dot(h, w2_ref[...], preferred_element_type=jnp.float32) + b2_ref[...]
    m = jnp.max(logits, axis=-1, keepdims=True)
    lse = m + jnp.log(jnp.sum(jnp.exp(logits - m), axis=-1, keepdims=True))
    out_ref[...] = logits - lse


def _tc_final(pp, s2g, gl, bel, w1, b1, w2, b2):
    return pl.pallas_call(
        _tc_final_body,
        out_shape=jax.ShapeDtypeStruct((G, C), jnp.float32),
    )(pp[0], pp[1], pp[2], s2g, gl[None, :], bel[None, :], w1, b1[None, :],
      w2, b2[None, :])


def kernel(x, edge_index, node_to_subgraph, subgraph_to_graph, batch,
           W1, b1, W2, b2, W3, b3, g1, g2, g3, gl, be1, be2, be3, bel,
           lin1_W, lin1_b, lin2_W, lin2_b):
    ei = edge_index.astype(jnp.int32).reshape(2, NW, ENCH, ECH)
    ed = ei.transpose(1, 2, 0, 3)
    dst = ei[1]
    n2s = node_to_subgraph.astype(jnp.int32)
    s2g = subgraph_to_graph.astype(jnp.int32)

    ones_deg = jnp.ones((ECH, DEGW), jnp.float32)
    z_deg = jnp.zeros((ECH, DEGW), jnp.float32)
    z_prop = jnp.zeros((ECH, H), jnp.float32)
    z_pool = jnp.zeros((SPAD // NS, H), jnp.float32)

    degp = _deg_sc(dst, ones_deg, z_deg)
    u1, dinv = _tc_pre(x, W1, degp)
    t1 = _prop_sc(u1, ed, z_prop)
    x1, u2 = _tc_mid(t1, u1, dinv, b1, g1, be1, W2)
    t2 = _prop_sc(u2, ed, z_prop)
    x2, u3 = _tc_mid(t2, u2, dinv, b2, g2, be2, W3)
    t3 = _prop_sc(u3, ed, z_prop)
    x3 = _tc_last(t3, u3, dinv, b3, g3, be3)

    zrow = jnp.zeros((NPAD - N, H), jnp.float32)
    n2sp = jnp.concatenate([n2s, jnp.zeros((NPAD - N,), jnp.int32)])
    pp = _pool_sc(jnp.concatenate([x1, zrow], axis=0),
                  jnp.concatenate([x2, zrow], axis=0),
                  jnp.concatenate([x3, zrow], axis=0),
                  n2sp, z_pool)
    return _tc_final(pp, s2g, gl, bel, lin1_W, lin1_b, lin2_W, lin2_b)
